# Initial kernel scaffold; baseline (speedup 1.0000x reference)
import jax
import jax.numpy as jnp
from jax.experimental import pallas as pl


def kernel(x, edge_index, edge_attr, params, Ws, bs):
    raise NotImplementedError("write your pallas kernel here")


# speedup vs baseline: 17.0967x; 17.0967x over previous
import jax
import jax.numpy as jnp
from jax.experimental import pallas as pl

_DEFAULT = jax.lax.Precision.DEFAULT
_HIGHEST = jax.lax.Precision.HIGHEST


def _dot(a, b, precision):
    return jax.lax.dot_general(
        a, b, (((1,), (0,)), ((), ())), preferred_element_type=jnp.float32,
        precision=precision,
    )


def _dot_t(a, b, precision):
    return jax.lax.dot_general(
        a, b, (((1,), (1,)), ((), ())), preferred_element_type=jnp.float32,
        precision=precision,
    )


def _fused_kernel(
    x_ref, src_ref, dst_ref, dst_row_ref, ea_ref,
    W0, As0, Ad0, We0, Ae0, b0,
    W1, As1, Ad1, We1, Ae1, b1,
    W2, As2, Ad2, We2, Ae2, b2,
    W3, As3, Ad3, We3, Ae3, b3,
    Ws_ref, bs_ref, out_ref,
):
    N = x_ref.shape[0]
    E = src_ref.shape[0]

    src = src_ref[...]
    dst = dst_ref[...]
    dst_row = dst_row_ref[...]

    iota_en = jax.lax.broadcasted_iota(jnp.int32, (E, N), 1)
    iota_ne = jax.lax.broadcasted_iota(jnp.int32, (N, E), 0)
    src_oh = (iota_en == src).astype(jnp.float32)
    dst_oh = (iota_en == dst).astype(jnp.float32)
    dst_mask = iota_ne == dst_row
    ea = ea_ref[...]

    h = x_ref[...]
    layers = (
        (W0, As0, Ad0, We0, Ae0, b0),
        (W1, As1, Ad1, We1, Ae1, b1),
        (W2, As2, Ad2, We2, Ae2, b2),
        (W3, As3, Ad3, We3, Ae3, b3),
    )
    for i, (W, As, Ad, We, Ae, b) in enumerate(layers):
        hp = _dot(h, W[...], _DEFAULT)
        a_src = jnp.sum(hp * As[...], axis=1, keepdims=True)
        a_dst = jnp.sum(hp * Ad[...], axis=1, keepdims=True)
        ep = _dot(ea, We[...], _DEFAULT)
        a_edge = jnp.sum(ep * Ae[...], axis=1, keepdims=True)
        alpha = (_dot(src_oh, a_src, _HIGHEST)
                 + _dot(dst_oh, a_dst, _HIGHEST) + a_edge)
        alpha = jnp.where(alpha >= 0, alpha, 0.2 * alpha)
        alpha_row = jnp.where(dst_mask, jnp.broadcast_to(alpha.T, (N, E)), -1e30)
        m = jnp.max(alpha_row, axis=1, keepdims=True)
        m = jnp.where(m < -1e29, 0.0, m)
        ex = jnp.exp(alpha - _dot(dst_oh, m, _HIGHEST))
        denom = _dot(dst_mask.astype(jnp.float32), ex, _HIGHEST)
        coef = ex / (_dot(dst_oh, denom, _HIGHEST) + 1e-16)
        a_eff = _dot(
            jnp.where(dst_mask, jnp.broadcast_to(coef.T, (N, E)), 0.0),
            src_oh, _HIGHEST,
        )
        h = _dot(a_eff, hp, _HIGHEST) + b[...]
        if i < 3:
            h = jnp.maximum(h, 0.0)

    conn = _dot_t(h, h, _DEFAULT)
    logits = _dot(conn, Ws_ref[...], _DEFAULT) + bs_ref[...]
    out_ref[...] = jax.lax.logistic(logits)


def kernel(x, edge_index, edge_attr, params, Ws, bs):
    N = x.shape[0]
    E = edge_index.shape[1]
    src = edge_index[0].reshape(E, 1)
    dst = edge_index[1].reshape(E, 1)
    dst_row = edge_index[1].reshape(1, E)

    flat = []
    for (W, As, Ad, We, Ae, b) in params:
        d = W.shape[1]
        flat.extend([W, As.reshape(1, d), Ad.reshape(1, d),
                     We, Ae.reshape(1, d), b.reshape(1, d)])

    return pl.pallas_call(
        _fused_kernel,
        out_shape=jax.ShapeDtypeStruct((N, 1), jnp.float32),
    )(x, src, dst, dst_row, edge_attr, *flat, Ws, bs.reshape(1, 1))
